# baseline (speedup 1.0000x reference)
import functools

import jax
import jax.numpy as jnp
from jax import lax
from jax.experimental import pallas as pl
from jax.experimental.pallas import tpu as pltpu
from jax.experimental.pallas import tpu_sc as plsc

_VOCAB = 100000
_B = 1024
_T = 200
_D = 512
_N = _B * _T
_NW = 32
_PER_W = _N // _NW
_C = 16
_NCH = _PER_W // _C
_LANES = 16


def _pe_table():
    x = jnp.arange(_T, dtype=jnp.float32)[:, None]
    y = jnp.arange(_D, dtype=jnp.float32)[None, :]
    temp = jnp.power(10000.0, 2.0 * y / _D).astype(jnp.float32)
    s = jnp.sin(x / temp)
    c = jnp.cos(x / temp)
    z = jnp.zeros((_T, _D), dtype=jnp.float32)
    z = z.at[:, 0::2].set(s[:, 0::2])
    z = z.at[:, 1::2].set(c[:, 1::2])
    return z


def _sc_body(table_hbm, idx_hbm, pe_hbm, out_hbm,
             idx_v, pe_v, buf0, buf1,
             in_sem0, in_sem1, out_sem0, out_sem1):
    cid = lax.axis_index("c")
    sid = lax.axis_index("s")
    wid = sid * 2 + cid
    base = wid * _PER_W

    bufs = (buf0, buf1)
    in_sems = (in_sem0, in_sem1)
    out_sems = (out_sem0, out_sem1)

    pltpu.sync_copy(idx_hbm.at[pl.ds(base, _PER_W)], idx_v)
    pltpu.sync_copy(pe_hbm, pe_v)

    def start_in(c, b):
        pltpu.make_async_copy(
            table_hbm.at[idx_v.at[pl.ds(c * _C, _C)]], bufs[b], in_sems[b]
        ).start()

    def wait_in(b):
        pltpu.make_async_copy(
            table_hbm.at[idx_v.at[pl.ds(0, _C)]], bufs[b], in_sems[b]
        ).wait()

    def start_out(c, b):
        pltpu.make_async_copy(
            bufs[b], out_hbm.at[wid * _NCH + c], out_sems[b]
        ).start()

    def wait_out(b):
        pltpu.make_async_copy(
            bufs[b], out_hbm.at[0], out_sems[b]
        ).wait()

    start_in(0, 0)
    start_in(1, 1)

    @pl.loop(0, _NCH, step=2)
    def _chunks(c0):
        for b in range(2):
            c = c0 + b
            wait_in(b)
            buf = bufs[b]

            @pl.loop(0, _C)
            def _rows(r):
                t = lax.rem(c * _C + r, _T)
                for k in range(_D // _LANES):
                    sl = pl.ds(k * _LANES, _LANES)
                    buf[r, sl] = buf[r, sl] + pe_v[t, sl]

            start_out(c, b)

            @pl.when(c + 2 < _NCH)
            def _prefetch():
                wait_out(b)
                start_in(c + 2, b)

    wait_out(0)
    wait_out(1)


@functools.partial(jax.jit, static_argnums=())
def _run(table, idx, pe):
    grid_kernel = pl.kernel(
        _sc_body,
        out_type=jax.ShapeDtypeStruct((_N // _C, _C, _D), jnp.float32),
        mesh=plsc.VectorSubcoreMesh(core_axis_name="c", subcore_axis_name="s"),
        scratch_types=[
            pltpu.VMEM((_PER_W,), jnp.int32),
            pltpu.VMEM((_T, _D), jnp.float32),
            pltpu.VMEM((_C, _D), jnp.float32),
            pltpu.VMEM((_C, _D), jnp.float32),
            pltpu.SemaphoreType.DMA,
            pltpu.SemaphoreType.DMA,
            pltpu.SemaphoreType.DMA,
            pltpu.SemaphoreType.DMA,
        ],
    )
    return grid_kernel(table, idx, pe)


def kernel(X, table):
    idx = X.reshape(-1).astype(jnp.int32)
    pe = _pe_table()
    out = _run(table, idx, pe)
    return out.reshape(_B, _T, _D)


# speedup vs baseline: 1.0112x; 1.0112x over previous
import functools

import jax
import jax.numpy as jnp
from jax import lax
from jax.experimental import pallas as pl
from jax.experimental.pallas import tpu as pltpu
from jax.experimental.pallas import tpu_sc as plsc

_VOCAB = 100000
_B = 1024
_T = 200
_D = 512
_NG = 16
_SEQ_PER_G = _B // _NG
_NBLK = _T // 8
_BPW = 13
_C = 8
_NBUF = 8
_LANES = 16


def _pe_table():
    x = jnp.arange(_T, dtype=jnp.float32)[:, None]
    y = jnp.arange(_D, dtype=jnp.float32)[None, :]
    temp = jnp.power(10000.0, 2.0 * y / _D).astype(jnp.float32)
    s = jnp.sin(x / temp)
    c = jnp.cos(x / temp)
    z = jnp.zeros((_T, _D), dtype=jnp.float32)
    z = z.at[:, 0::2].set(s[:, 0::2])
    z = z.at[:, 1::2].set(c[:, 1::2])
    return z


def _split_parity_pe():
    pe3 = _pe_table().reshape(_NBLK, _C, _D)
    even = pe3[0::2]
    odd = jnp.concatenate([pe3[1::2],
                           jnp.zeros((1, _C, _D), jnp.float32)])
    return jnp.stack([even, odd]).reshape(2, _BPW * _C, _D)


def _sc_body(table_hbm, idx_hbm, pe_hbm, out_hbm, idx_v, pe_v, *rest):
    bufs = rest[:_NBUF]
    in_sems = rest[_NBUF:2 * _NBUF]
    out_sems = rest[2 * _NBUF:3 * _NBUF]

    h = lax.axis_index("c")
    g = lax.axis_index("s")
    nblk = _BPW - h
    total = _SEQ_PER_G * nblk

    pltpu.sync_copy(idx_hbm.at[h, g], idx_v)
    pltpu.sync_copy(pe_hbm.at[h], pe_v)

    def start_in(ci, b):
        sl = lax.div(ci, nblk)
        bi = ci - sl * nblk
        off = (sl * _BPW + bi) * _C
        pltpu.make_async_copy(
            table_hbm.at[idx_v.at[pl.ds(off, _C)]], bufs[b], in_sems[b]
        ).start()

    def wait_in(b):
        pltpu.make_async_copy(
            table_hbm.at[idx_v.at[pl.ds(0, _C)]], bufs[b], in_sems[b]
        ).wait()

    def start_out(ci, b):
        sl = lax.div(ci, nblk)
        bi = ci - sl * nblk
        blk = (g * _SEQ_PER_G + sl) * _NBLK + 2 * bi + h
        pltpu.make_async_copy(
            bufs[b], out_hbm.at[blk], out_sems[b]
        ).start()

    def wait_out(b):
        pltpu.make_async_copy(
            bufs[b], out_hbm.at[0], out_sems[b]
        ).wait()

    def add_pe(ci, b):
        sl = lax.div(ci, nblk)
        bi = ci - sl * nblk
        buf = bufs[b]

        @pl.loop(0, _C)
        def _rows(r):
            t = bi * _C + r
            for k in range(_D // _LANES):
                sl_ = pl.ds(k * _LANES, _LANES)
                plsc.addupdate(buf.at[r, sl_], pe_v[t, sl_])

    for b in range(_NBUF):
        start_in(b, b)

    @pl.loop(0, total, step=_NBUF)
    def _chunks(c0):
        for b in range(_NBUF):
            ci = c0 + b
            wait_in(b)
            add_pe(ci, b)
            start_out(ci, b)

            @pl.when(ci + _NBUF < total)
            def _prefetch():
                wait_out(b)
                start_in(ci + _NBUF, b)

    for b in range(_NBUF):
        wait_out(b)


@functools.partial(jax.jit, static_argnums=())
def _run(table, idx, pe):
    grid_kernel = pl.kernel(
        _sc_body,
        out_type=jax.ShapeDtypeStruct((_B * _NBLK, _C, _D), jnp.float32),
        mesh=plsc.VectorSubcoreMesh(core_axis_name="c", subcore_axis_name="s"),
        scratch_types=[
            pltpu.VMEM((_SEQ_PER_G * _BPW * _C,), jnp.int32),
            pltpu.VMEM((_BPW * _C, _D), jnp.float32),
        ] + [pltpu.VMEM((_C, _D), jnp.float32)] * _NBUF
          + [pltpu.SemaphoreType.DMA] * (2 * _NBUF),
    )
    return grid_kernel(table, idx, pe)


def kernel(X, table):
    x3 = X.reshape(_B, _NBLK, _C).astype(jnp.int32)
    even = x3[:, 0::2]
    odd = jnp.concatenate(
        [x3[:, 1::2], jnp.zeros((_B, 1, _C), jnp.int32)], axis=1)
    idx = jnp.stack([even, odd]).reshape(2, _NG, _SEQ_PER_G * _BPW * _C)
    pe = _split_parity_pe()
    out = _run(table, idx, pe)
    return out.reshape(_B, _T, _D)
